# baseline (speedup 1.0000x reference)
import functools

import numpy as np

import jax
import jax.numpy as jnp
from jax import lax
from jax.experimental import pallas as pl
from jax.experimental.pallas import tpu as pltpu
from jax.experimental.pallas import tpu_sc as plsc

_NC = 2
_NS = 16
_NW = _NC * _NS
_CHUNK = 128
_NB = 6
_P = 3
_DP = 40
_FT = 8
_LANE = 128


def _build(NT, n_chunks, D, dtype):
    nft = 2 * D // _FT
    nft_tab = D // _FT
    slab = n_chunks * _CHUNK
    mesh = plsc.VectorSubcoreMesh(core_axis_name="c", subcore_axis_name="s")
    out_type = jax.ShapeDtypeStruct((nft, NT, _FT, _LANE), dtype)

    @functools.partial(
        pl.kernel,
        mesh=mesh,
        out_type=out_type,
        compiler_params=pltpu.CompilerParams(use_tc_tiling_on_sc=False,
                                             needs_layout_passes=False),
        scratch_types=[
            pltpu.VMEM((slab,), jnp.int32),
            pltpu.VMEM((slab,), jnp.int32),
            pltpu.VMEM((_NB, 2 * _CHUNK, _DP), dtype),
            pltpu.VMEM((_NB, nft, _FT, _LANE), dtype),
            pltpu.SemaphoreType.DMA,
            pltpu.SemaphoreType.DMA((_NB,)),
            pltpu.SemaphoreType.DMA((_NB,)),
        ],
    )
    def k(idx_a_hbm, idx_r_hbm, w_atom_hbm, w_res_hbm, out_hbm,
          idx_a_v, idx_r_v, rows, tiles, sem_i, sem_g, sem_w):
        wid = lax.axis_index("s") * _NC + lax.axis_index("c")
        base_t = jnp.minimum(wid * n_chunks, NT - n_chunks)
        base = base_t * _CHUNK

        pltpu.async_copy(idx_a_hbm.at[pl.ds(base, slab)], idx_a_v, sem_i)
        pltpu.async_copy(idx_r_hbm.at[pl.ds(base, slab)], idx_r_v, sem_i)
        pltpu.make_async_copy(idx_a_hbm.at[pl.ds(0, slab)], idx_a_v, sem_i).wait()
        pltpu.make_async_copy(idx_r_hbm.at[pl.ds(0, slab)], idx_r_v, sem_i).wait()

        def gather(j, b):
            sl = pl.ds(j * _CHUNK, _CHUNK)
            pltpu.async_copy(w_atom_hbm.at[idx_a_v.at[sl]],
                             rows.at[b, pl.ds(0, _CHUNK)], sem_g.at[b])
            pltpu.async_copy(w_res_hbm.at[idx_r_v.at[sl]],
                             rows.at[b, pl.ds(_CHUNK, _CHUNK)], sem_g.at[b])

        def wait_gather(b):
            pltpu.make_async_copy(w_atom_hbm.at[idx_a_v.at[pl.ds(0, _CHUNK)]],
                                  rows.at[b, pl.ds(0, _CHUNK)], sem_g.at[b]).wait()
            pltpu.make_async_copy(w_res_hbm.at[idx_r_v.at[pl.ds(0, _CHUNK)]],
                                  rows.at[b, pl.ds(_CHUNK, _CHUNK)], sem_g.at[b]).wait()

        def write(j, b):
            pltpu.async_copy(tiles.at[b], out_hbm.at[:, base_t + j], sem_w.at[b])

        def wait_write(b):
            pltpu.make_async_copy(tiles.at[b], out_hbm.at[:, 0], sem_w.at[b]).wait()

        iota = lax.iota(jnp.int32, 16)

        def transpose(b):
            rb = rows.at[b]
            tb = tiles.at[b]
            for ft in range(nft):
                ro = (ft // nft_tab) * _CHUNK
                f0 = (ft % nft_tab) * _FT
                for r in range(_FT):
                    col = jnp.full((16,), f0 + r, jnp.int32)
                    vs = [plsc.load_gather(rb, [ro + g * 16 + iota, col])
                          for g in range(_CHUNK // 16)]
                    for g, v in enumerate(vs):
                        tb[ft, r, pl.ds(g * 16, 16)] = v

        for b in range(_P):
            gather(b, b)

        def body(j, carry):
            b = lax.rem(j, _NB)
            wait_gather(b)
            bp = lax.rem(j + _P, _NB)

            @pl.when(j + _P < n_chunks)
            def _():
                gather(j + _P, bp)

            @pl.when(j >= _NB)
            def _():
                wait_write(b)

            transpose(b)
            write(j, b)
            return carry

        lax.fori_loop(0, n_chunks, body, 0)

        for b in range(_NB):
            wait_write(b)

    return k


def kernel(atom_type, residue_type, W_atom, W_res):
    B = atom_type.shape[0]
    D = W_atom.shape[1]
    NT = -(-B // _CHUNK)
    n_chunks = -(-NT // _NW)
    n_chunks = -(-n_chunks // _NB) * _NB
    B_pad = NT * _CHUNK

    idx_a = jnp.zeros((B_pad,), jnp.int32).at[:B].set(atom_type.astype(jnp.int32))
    idx_r = jnp.zeros((B_pad,), jnp.int32).at[:B].set(residue_type.astype(jnp.int32))

    W_a = jnp.pad(W_atom, ((0, 0), (0, _DP - D)))
    W_r = jnp.pad(W_res, ((0, 0), (0, _DP - D)))

    out4 = _build(NT, n_chunks, D, W_atom.dtype)(idx_a, idx_r, W_a, W_r)
    return out4.transpose(1, 3, 0, 2).reshape(B_pad, 2 * D)[:B]


# speedup vs baseline: 1.1072x; 1.1072x over previous
import functools

import numpy as np

import jax
import jax.numpy as jnp
from jax import lax
from jax.experimental import pallas as pl
from jax.experimental.pallas import tpu as pltpu
from jax.experimental.pallas import tpu_sc as plsc

_NC = 2
_NS = 16
_NW = _NC * _NS
_CHUNK = 128
_NB = 6
_P = 3
_DP = 40
_FT = 8
_LANE = 128


def _build(NT, n_chunks, D, dtype):
    nft = 2 * D // _FT
    nft_tab = D // _FT
    slab = n_chunks * _CHUNK
    mesh = plsc.VectorSubcoreMesh(core_axis_name="c", subcore_axis_name="s")
    out_type = jax.ShapeDtypeStruct((nft, NT, _FT, _LANE), dtype)

    @functools.partial(
        pl.kernel,
        mesh=mesh,
        out_type=out_type,
        compiler_params=pltpu.CompilerParams(use_tc_tiling_on_sc=False,
                                             needs_layout_passes=False),
        scratch_types=[
            pltpu.VMEM((slab,), jnp.int32),
            pltpu.VMEM((slab,), jnp.int32),
            pltpu.VMEM((_NB, 2 * _CHUNK, _DP), dtype),
            pltpu.VMEM((_NB, nft, _FT, _LANE), dtype),
            pltpu.SemaphoreType.DMA,
            pltpu.SemaphoreType.DMA((_NB,)),
            pltpu.SemaphoreType.DMA((_NB,)),
        ],
    )
    def k(idx_a_hbm, idx_r_hbm, w_atom_hbm, w_res_hbm, out_hbm,
          idx_a_v, idx_r_v, rows, tiles, sem_i, sem_g, sem_w):
        wid = lax.axis_index("s") * _NC + lax.axis_index("c")
        base_t = jnp.minimum(wid * n_chunks, NT - n_chunks)
        base = base_t * _CHUNK

        pltpu.async_copy(idx_a_hbm.at[pl.ds(base, slab)], idx_a_v, sem_i)
        pltpu.async_copy(idx_r_hbm.at[pl.ds(base, slab)], idx_r_v, sem_i)
        pltpu.make_async_copy(idx_a_hbm.at[pl.ds(0, slab)], idx_a_v, sem_i).wait()
        pltpu.make_async_copy(idx_r_hbm.at[pl.ds(0, slab)], idx_r_v, sem_i).wait()

        def gather(j, b):
            sl = pl.ds(j * _CHUNK, _CHUNK)
            pltpu.async_copy(w_atom_hbm.at[idx_a_v.at[sl]],
                             rows.at[b, pl.ds(0, _CHUNK)], sem_g.at[b])
            pltpu.async_copy(w_res_hbm.at[idx_r_v.at[sl]],
                             rows.at[b, pl.ds(_CHUNK, _CHUNK)], sem_g.at[b])

        def wait_gather(b):
            pltpu.make_async_copy(w_atom_hbm.at[idx_a_v.at[pl.ds(0, _CHUNK)]],
                                  rows.at[b, pl.ds(0, _CHUNK)], sem_g.at[b]).wait()
            pltpu.make_async_copy(w_res_hbm.at[idx_r_v.at[pl.ds(0, _CHUNK)]],
                                  rows.at[b, pl.ds(_CHUNK, _CHUNK)], sem_g.at[b]).wait()

        def write(j, b):
            pltpu.async_copy(tiles.at[b], out_hbm.at[:, base_t + j], sem_w.at[b])

        def wait_write(b):
            pltpu.make_async_copy(tiles.at[b], out_hbm.at[:, 0], sem_w.at[b]).wait()

        iota = lax.iota(jnp.int32, 16)

        def transpose(b):
            rb = rows.at[b]
            tb = tiles.at[b]
            for ft in range(nft):
                ro = (ft // nft_tab) * _CHUNK
                f0 = (ft % nft_tab) * _FT
                for r in range(_FT):
                    col = jnp.full((16,), f0 + r, jnp.int32)
                    vs = [plsc.load_gather(rb, [ro + g * 16 + iota, col])
                          for g in range(_CHUNK // 16)]
                    for g, v in enumerate(vs):
                        tb[ft, r, pl.ds(g * 16, 16)] = v

        for b in range(_P):
            gather(b, b)

        def body(j, carry):
            b = lax.rem(j, _NB)
            wait_gather(b)
            bp = lax.rem(j + _P, _NB)

            @pl.when(j + _P < n_chunks)
            def _():
                gather(j + _P, bp)

            @pl.when(j >= _NB)
            def _():
                wait_write(b)

            transpose(b)
            write(j, b)
            return carry

        lax.fori_loop(0, n_chunks, body, 0)

        for b in range(_NB):
            wait_write(b)

    return k


def kernel(atom_type, residue_type, W_atom, W_res):
    B = atom_type.shape[0]
    D = W_atom.shape[1]
    NT = -(-B // _CHUNK)
    n_chunks = -(-NT // _NW)
    B_pad = NT * _CHUNK

    idx_a = jnp.zeros((B_pad,), jnp.int32).at[:B].set(atom_type.astype(jnp.int32))
    idx_r = jnp.zeros((B_pad,), jnp.int32).at[:B].set(residue_type.astype(jnp.int32))

    W_a = jnp.pad(W_atom, ((0, 0), (0, _DP - D)))
    W_r = jnp.pad(W_res, ((0, 0), (0, _DP - D)))

    out4 = _build(NT, n_chunks, D, W_atom.dtype)(idx_a, idx_r, W_a, W_r)
    return out4.transpose(1, 3, 0, 2).reshape(B_pad, 2 * D)[:B]
